# Initial kernel scaffold; baseline (speedup 1.0000x reference)
import jax
import jax.numpy as jnp
from jax.experimental import pallas as pl


def kernel(tokens, pos_indices, pos_table):
    raise NotImplementedError("write your pallas kernel here")


# speedup vs baseline: 1.0340x; 1.0340x over previous
import functools

import jax
import jax.numpy as jnp
from jax import lax
from jax.experimental import pallas as pl
from jax.experimental.pallas import tpu as pltpu
from jax.experimental.pallas import tpu_sc as plsc

_NC = 2
_NS = 16
_NW = _NC * _NS
_LANES = 16


@functools.partial(jax.jit, static_argnames=("rows", "d", "chunk"))
def _pos_enc_sc(tok, idx, table, *, rows, d, chunk):
    rpw = rows // _NW
    nchunk = rpw // chunk
    nvec = d // _LANES

    mesh = plsc.VectorSubcoreMesh(core_axis_name="c", subcore_axis_name="s")

    @functools.partial(
        pl.kernel,
        mesh=mesh,
        out_type=jax.ShapeDtypeStruct((rows, d), jnp.float32),
        scratch_types=[
            pltpu.VMEM((chunk,), jnp.int32),
            pltpu.VMEM((chunk, d), jnp.float32),
            pltpu.VMEM((chunk, d), jnp.float32),
            pltpu.SemaphoreType.DMA,
        ],
    )
    def k(tok_hbm, idx_hbm, tab_hbm, out_hbm, idx_v, buf_v, row_v, sem):
        wid = lax.axis_index("s") * _NC + lax.axis_index("c")
        base = wid * rpw

        def chunk_body(g, carry):
            r0 = base + g * chunk
            pltpu.sync_copy(idx_hbm.at[pl.ds(r0, chunk)], idx_v)
            pltpu.sync_copy(tok_hbm.at[pl.ds(r0, chunk)], buf_v)
            pltpu.async_copy(tab_hbm.at[idx_v], row_v, sem).wait()

            def add_row(r, c2):
                for j in range(nvec):
                    sl = pl.ds(j * _LANES, _LANES)
                    plsc.addupdate(buf_v.at[r, sl], row_v[r, sl])
                return c2

            lax.fori_loop(0, chunk, add_row, 0)
            pltpu.sync_copy(buf_v, out_hbm.at[pl.ds(r0, chunk)])
            return carry

        lax.fori_loop(0, nchunk, chunk_body, 0)

    return k(tok, idx, table)


def kernel(tokens, pos_indices, pos_table):
    b, s, d = tokens.shape
    rows = b * s
    tok = tokens.reshape(rows, d)
    idx = pos_indices.reshape(rows).astype(jnp.int32)
    out = _pos_enc_sc(tok, idx, pos_table, rows=rows, d=d, chunk=32)
    return out.reshape(b, s, d)


# speedup vs baseline: 1.4841x; 1.4353x over previous
import functools

import jax
import jax.numpy as jnp
from jax import lax
from jax.experimental import pallas as pl
from jax.experimental.pallas import tpu as pltpu
from jax.experimental.pallas import tpu_sc as plsc

_NC = 2
_NS = 16
_NW = _NC * _NS
_LANES = 16


@functools.partial(jax.jit, static_argnames=("rows", "d", "chunk"))
def _pos_enc_sc(tok, idx, table, *, rows, d, chunk):
    rpw = rows // _NW
    nchunk = rpw // chunk
    half = nchunk // 2
    nvec = d // _LANES

    mesh = plsc.VectorSubcoreMesh(core_axis_name="c", subcore_axis_name="s")

    @functools.partial(
        pl.kernel,
        mesh=mesh,
        out_type=jax.ShapeDtypeStruct((rows, d), jnp.float32),
        scratch_types=[
            pltpu.VMEM((rpw,), jnp.int32),
            pltpu.VMEM((2, chunk, d), jnp.float32),
            pltpu.VMEM((2, chunk, d), jnp.float32),
            pltpu.VMEM((2, chunk, d), jnp.float32),
            pltpu.SemaphoreType.DMA,
            pltpu.SemaphoreType.DMA,
            pltpu.SemaphoreType.DMA,
            pltpu.SemaphoreType.DMA,
            pltpu.SemaphoreType.DMA,
            pltpu.SemaphoreType.DMA,
        ],
    )
    def k(tok_hbm, idx_hbm, tab_hbm, out_hbm, idx_v, tok_v, row_v, sum_v,
          ts0, ts1, rs0, rs1, os0, os1):
        wid = lax.axis_index("s") * _NC + lax.axis_index("c")
        base = wid * rpw
        tsem = (ts0, ts1)
        rsem = (rs0, rs1)
        osem = (os0, os1)

        pltpu.sync_copy(idx_hbm.at[pl.ds(base, rpw)], idx_v)

        def start_in(g, slot):
            r0 = base + g * chunk
            pltpu.async_copy(tok_hbm.at[pl.ds(r0, chunk)], tok_v.at[slot],
                             tsem[slot])
            pltpu.async_copy(tab_hbm.at[idx_v.at[pl.ds(g * chunk, chunk)]],
                             row_v.at[slot], rsem[slot])

        def wait_in(g, slot):
            r0 = base + g * chunk
            pltpu.make_async_copy(tok_hbm.at[pl.ds(r0, chunk)],
                                  tok_v.at[slot], tsem[slot]).wait()
            pltpu.make_async_copy(tab_hbm.at[idx_v.at[pl.ds(g * chunk, chunk)]],
                                  row_v.at[slot], rsem[slot]).wait()

        def start_out(g, slot):
            r0 = base + g * chunk
            pltpu.async_copy(sum_v.at[slot], out_hbm.at[pl.ds(r0, chunk)],
                             osem[slot])

        def wait_out(g, slot):
            r0 = base + g * chunk
            pltpu.make_async_copy(sum_v.at[slot],
                                  out_hbm.at[pl.ds(r0, chunk)],
                                  osem[slot]).wait()

        def add_chunk(slot):
            def add_row(r, c2):
                for j in range(nvec):
                    sl = pl.ds(j * _LANES, _LANES)
                    sum_v[slot, r, sl] = tok_v[slot, r, sl] + row_v[slot, r, sl]
                return c2
            lax.fori_loop(0, chunk, add_row, 0)

        start_in(0, 0)
        start_in(1, 1)

        def step(t, carry):
            for slot in (0, 1):
                g = 2 * t + slot
                wait_in(g, slot)

                @pl.when(t > 0)
                def _():
                    wait_out(g - 2, slot)

                add_chunk(slot)
                start_out(g, slot)

                @pl.when(t < half - 1)
                def _():
                    start_in(g + 2, slot)
            return carry

        lax.fori_loop(0, half, step, 0)
        wait_out(nchunk - 2, 0)
        wait_out(nchunk - 1, 1)

    return k(tok, idx, table)


def kernel(tokens, pos_indices, pos_table):
    b, s, d = tokens.shape
    rows = b * s
    tok = tokens.reshape(rows, d)
    idx = pos_indices.reshape(rows).astype(jnp.int32)
    out = _pos_enc_sc(tok, idx, pos_table, rows=rows, d=d, chunk=16)
    return out.reshape(b, s, d)


# speedup vs baseline: 1.9233x; 1.2960x over previous
import functools

import jax
import jax.numpy as jnp
from jax import lax
from jax.experimental import pallas as pl
from jax.experimental.pallas import tpu as pltpu
from jax.experimental.pallas import tpu_sc as plsc

_NC = 2
_NS = 16
_NW = _NC * _NS
_LANES = 16
_SLOTS = 4


@functools.partial(jax.jit, static_argnames=("rows", "d", "chunk"))
def _pos_enc_sc(tok, idx, table, *, rows, d, chunk):
    rpw = rows // _NW
    nchunk = rpw // chunk
    nstep = nchunk // _SLOTS
    nvec = d // _LANES

    mesh = plsc.VectorSubcoreMesh(core_axis_name="c", subcore_axis_name="s")

    @functools.partial(
        pl.kernel,
        mesh=mesh,
        out_type=jax.ShapeDtypeStruct((rows, d), jnp.float32),
        scratch_types=[
            pltpu.VMEM((rpw,), jnp.int32),
            pltpu.VMEM((_SLOTS, chunk, d), jnp.float32),
            pltpu.VMEM((_SLOTS, chunk, d), jnp.float32),
            [pltpu.SemaphoreType.DMA] * _SLOTS,
            [pltpu.SemaphoreType.DMA] * _SLOTS,
            [pltpu.SemaphoreType.DMA] * _SLOTS,
        ],
    )
    def k(tok_hbm, idx_hbm, tab_hbm, out_hbm, idx_v, tok_v, row_v,
          tsem, rsem, osem):
        wid = lax.axis_index("s") * _NC + lax.axis_index("c")
        base = wid * rpw

        pltpu.sync_copy(idx_hbm.at[pl.ds(base, rpw)], idx_v)

        def start_in(g, slot):
            r0 = base + g * chunk
            pltpu.async_copy(tok_hbm.at[pl.ds(r0, chunk)], tok_v.at[slot],
                             tsem[slot])
            pltpu.async_copy(tab_hbm.at[idx_v.at[pl.ds(g * chunk, chunk)]],
                             row_v.at[slot], rsem[slot])

        def wait_in(g, slot):
            r0 = base + g * chunk
            pltpu.make_async_copy(tok_hbm.at[pl.ds(r0, chunk)],
                                  tok_v.at[slot], tsem[slot]).wait()
            pltpu.make_async_copy(tab_hbm.at[idx_v.at[pl.ds(g * chunk, chunk)]],
                                  row_v.at[slot], rsem[slot]).wait()

        def start_out(g, slot):
            r0 = base + g * chunk
            pltpu.async_copy(tok_v.at[slot], out_hbm.at[pl.ds(r0, chunk)],
                             osem[slot])

        def wait_out(g, slot):
            r0 = base + g * chunk
            pltpu.make_async_copy(tok_v.at[slot],
                                  out_hbm.at[pl.ds(r0, chunk)],
                                  osem[slot]).wait()

        def add_chunk(slot):
            def add_row(r, c2):
                for j in range(nvec):
                    sl = pl.ds(j * _LANES, _LANES)
                    plsc.addupdate(tok_v.at[slot, r, sl], row_v[slot, r, sl])
                return c2
            lax.fori_loop(0, chunk, add_row, 0)

        start_in(0, 0)
        start_in(1, 1)

        def step(t, carry):
            for u in range(_SLOTS):
                slot = u
                g = _SLOTS * t + u

                @pl.when(g >= 2)
                def _():
                    wait_out(g - 2, (u - 2) % _SLOTS)

                @pl.when(g + 2 < nchunk)
                def _():
                    start_in(g + 2, (u + 2) % _SLOTS)

                wait_in(g, slot)
                add_chunk(slot)
                start_out(g, slot)
            return carry

        lax.fori_loop(0, nstep, step, 0)
        wait_out(nchunk - 2, (nchunk - 2) % _SLOTS)
        wait_out(nchunk - 1, (nchunk - 1) % _SLOTS)

    return k(tok, idx, table)


def kernel(tokens, pos_indices, pos_table):
    b, s, d = tokens.shape
    rows = b * s
    tok = tokens.reshape(rows, d)
    idx = pos_indices.reshape(rows).astype(jnp.int32)
    out = _pos_enc_sc(tok, idx, pos_table, rows=rows, d=d, chunk=8)
    return out.reshape(b, s, d)
